# baseline (speedup 1.0000x reference)
import functools

import jax
import jax.numpy as jnp
from jax import lax
from jax.experimental import pallas as pl
from jax.experimental.pallas import tpu as pltpu

H = 16
HD = 64
LR = 0.1
C = 32
F32 = jnp.float32


BF16 = jnp.bfloat16


def _mm_body(x_ref, w_ref, o_ref):
    o_ref[...] = jnp.dot(x_ref[...], w_ref[...],
                         preferred_element_type=F32).astype(o_ref.dtype)


def _matmul(x, w, out_dtype, bm=1024, bn=1024):
    m, k = x.shape
    _, n = w.shape
    return pl.pallas_call(
        _mm_body,
        grid=(m // bm, n // bn),
        in_specs=[
            pl.BlockSpec((bm, k), lambda i, j: (i, 0)),
            pl.BlockSpec((k, bn), lambda i, j: (0, j)),
        ],
        out_specs=pl.BlockSpec((bm, bn), lambda i, j: (i, j)),
        out_shape=jax.ShapeDtypeStruct((m, n), out_dtype),
        compiler_params=pltpu.CompilerParams(
            dimension_semantics=("parallel", "parallel")),
        name="proj_mm",
    )(x, w)


def _solve_body(k_ref, v_ref, wx_ref, *, r, beta):
    rows = lax.broadcasted_iota(jnp.int32, (r, r), 0)
    cols = lax.broadcasted_iota(jnp.int32, (r, r), 1)
    strict = (cols >> 2) < (rows >> 2)
    ident = jnp.where(rows == cols, 1.0, 0.0)
    for jj in range(2):
        kk = k_ref[:, jj * HD:(jj + 1) * HD]
        vv = v_ref[:, jj * HD:(jj + 1) * HD]
        g = lax.dot_general(kk, kk, (((1,), (1,)), ((), ())),
                            preferred_element_type=F32)
        a = jnp.where(strict, beta * g, 0.0).astype(BF16)
        t = ident
        gsz, sh = 4, 2
        while gsz < r:
            tb = t.astype(BF16)
            u = jnp.dot(tb, a, preferred_element_type=F32)
            u = jnp.dot(u.astype(BF16), tb, preferred_element_type=F32)
            rg = rows >> sh
            cg = cols >> sh
            msub = ((rg & 1) == 1) & (cg == rg - 1)
            t = t - jnp.where(msub, u, 0.0)
            gsz, sh = gsz * 2, sh + 1
        tb = t.astype(BF16)
        wx_ref[:, jj * 2 * HD:jj * 2 * HD + HD] = jnp.dot(
            tb, vv, preferred_element_type=F32).astype(BF16)
        wx_ref[:, jj * 2 * HD + HD:(jj + 1) * 2 * HD] = jnp.dot(
            tb, kk, preferred_element_type=F32).astype(BF16)


def _sweep_body(wx_ref, q_ref, k_ref, o_ref, n_ref, *, r, nc, beta, hpc):
    c = pl.program_id(1)

    @pl.when(c == 0)
    def _():
        n_ref[...] = jnp.zeros_like(n_ref)

    rows = lax.broadcasted_iota(jnp.int32, (r, r), 0)
    cols = lax.broadcasted_iota(jnp.int32, (r, r), 1)
    incl = (cols >> 2) <= (rows >> 2)

    for j in range(hpc):
        pr, odd = j >> 1, j & 1
        w = wx_ref[pr * r:(pr + 1) * r, odd * 2 * HD:odd * 2 * HD + HD]
        x = wx_ref[pr * r:(pr + 1) * r, odd * 2 * HD + HD:(odd + 1) * 2 * HD]
        q = q_ref[:, j * HD:(j + 1) * HD]
        kk = k_ref[:, j * HD:(j + 1) * HD]
        n = n_ref[j]
        nb = n.astype(BF16)
        e = w.astype(F32) - jnp.dot(x, nb, preferred_element_type=F32)
        eb = e.astype(BF16)
        qk = lax.dot_general(q, kk, (((1,), (1,)), ((), ())),
                             preferred_element_type=F32)
        aq = jnp.where(incl, beta * qk, 0.0).astype(BF16)
        o_ref[:, j * HD:(j + 1) * HD] = (
            jnp.dot(q, nb, preferred_element_type=F32)
            + jnp.dot(aq, eb, preferred_element_type=F32)).astype(BF16)
        n_ref[j] = n + beta * lax.dot_general(
            kk, eb, (((0,), (0,)), ((), ())),
            preferred_element_type=F32)


def kernel(x, Wq, Wk, Wv, Wo):
    b, s, d = x.shape
    r = C * b
    nc = s // C
    beta = LR / b
    hpc = H // 2

    xt = x.transpose(1, 0, 2).reshape(s * b, d).astype(BF16)
    wqkv = jnp.concatenate([Wq.T, Wk.T, Wv.T], axis=1).astype(BF16)

    qkv = _matmul(xt, wqkv, BF16, bm=1024, bn=1024)

    solve = functools.partial(_solve_body, r=r, beta=beta)
    wx = pl.pallas_call(
        solve,
        grid=(H // 2, nc),
        in_specs=[
            pl.BlockSpec((r, 2 * HD), lambda p, c: (c, H // 2 + p)),
            pl.BlockSpec((r, 2 * HD), lambda p, c: (c, H + p)),
        ],
        out_specs=pl.BlockSpec((r, 4 * HD), lambda p, c: (c * (H // 2) + p, 0)),
        out_shape=jax.ShapeDtypeStruct((nc * (H // 2) * r, 4 * HD), BF16),
        compiler_params=pltpu.CompilerParams(
            dimension_semantics=("parallel", "parallel")),
        name="chunk_solve",
    )(qkv, qkv)

    sweep = functools.partial(_sweep_body, r=r, nc=nc, beta=beta, hpc=hpc)
    o = pl.pallas_call(
        sweep,
        grid=(2, nc),
        in_specs=[
            pl.BlockSpec((hpc // 2 * r, 4 * HD), lambda gg, c: (c * 2 + gg, 0)),
            pl.BlockSpec((r, hpc * HD), lambda gg, c: (c, gg)),
            pl.BlockSpec((r, hpc * HD), lambda gg, c: (c, 2 + gg)),
        ],
        out_specs=pl.BlockSpec((r, hpc * HD), lambda gg, c: (c, gg)),
        out_shape=jax.ShapeDtypeStruct((s * b, d), BF16),
        scratch_shapes=[pltpu.VMEM((hpc, HD, HD), F32)],
        compiler_params=pltpu.CompilerParams(
            dimension_semantics=("parallel", "arbitrary")),
        name="chunk_sweep",
    )(wx, qkv, qkv)

    out = _matmul(o, Wo.T.astype(BF16), F32, bm=1024, bn=1024)
    return out.reshape(s, b, d).transpose(1, 0, 2)


# speedup vs baseline: 1.0725x; 1.0725x over previous
import functools

import jax
import jax.numpy as jnp
from jax import lax
from jax.experimental import pallas as pl
from jax.experimental.pallas import tpu as pltpu

H = 16
HD = 64
LR = 0.1
C = 32
F32 = jnp.float32


BF16 = jnp.bfloat16


def _mm_body(x_ref, w_ref, o_ref):
    o_ref[...] = jnp.dot(x_ref[...], w_ref[...],
                         preferred_element_type=F32).astype(o_ref.dtype)


def _matmul(x, w, out_dtype, bm=1024, bn=1024):
    m, k = x.shape
    _, n = w.shape
    return pl.pallas_call(
        _mm_body,
        grid=(m // bm, n // bn),
        in_specs=[
            pl.BlockSpec((bm, k), lambda i, j: (i, 0)),
            pl.BlockSpec((k, bn), lambda i, j: (0, j)),
        ],
        out_specs=pl.BlockSpec((bm, bn), lambda i, j: (i, j)),
        out_shape=jax.ShapeDtypeStruct((m, n), out_dtype),
        compiler_params=pltpu.CompilerParams(
            dimension_semantics=("parallel", "parallel")),
        name="proj_mm",
    )(x, w)


def _solve_body(k_ref, v_ref, wx_ref, *, r, beta, cg_n):
    rows = lax.broadcasted_iota(jnp.int32, (r, r), 0)
    cols = lax.broadcasted_iota(jnp.int32, (r, r), 1)
    strict = (cols >> 2) < (rows >> 2)
    ident = jnp.where(rows == cols, 1.0, 0.0)
    for ci in range(cg_n):
        for jj in range(2):
            kk = k_ref[ci * r:(ci + 1) * r, jj * HD:(jj + 1) * HD]
            vv = v_ref[ci * r:(ci + 1) * r, jj * HD:(jj + 1) * HD]
            g = lax.dot_general(kk, kk, (((1,), (1,)), ((), ())),
                                preferred_element_type=F32)
            a = jnp.where(strict, beta * g, 0.0).astype(BF16)
            t = ident
            gsz, sh = 4, 2
            while gsz < r:
                tb = t.astype(BF16)
                u = jnp.dot(tb, a, preferred_element_type=F32)
                u = jnp.dot(u.astype(BF16), tb, preferred_element_type=F32)
                rg = rows >> sh
                cg = cols >> sh
                msub = ((rg & 1) == 1) & (cg == rg - 1)
                t = t - jnp.where(msub, u, 0.0)
                gsz, sh = gsz * 2, sh + 1
            tb = t.astype(BF16)
            wx_ref[ci * r:(ci + 1) * r,
                   jj * 2 * HD:jj * 2 * HD + HD] = jnp.dot(
                tb, vv, preferred_element_type=F32).astype(BF16)
            wx_ref[ci * r:(ci + 1) * r,
                   jj * 2 * HD + HD:(jj + 1) * 2 * HD] = jnp.dot(
                tb, kk, preferred_element_type=F32).astype(BF16)


def _sweep_body(wx0, wx1, wx2, wx3, q_ref, k_ref, o_ref, n_ref,
                *, r, beta, hpc, cb_n):
    c = pl.program_id(1)

    @pl.when(c == 0)
    def _():
        n_ref[...] = jnp.zeros_like(n_ref)

    rows = lax.broadcasted_iota(jnp.int32, (r, r), 0)
    cols = lax.broadcasted_iota(jnp.int32, (r, r), 1)
    incl = (cols >> 2) <= (rows >> 2)
    wxr = (wx0, wx1, wx2, wx3)

    for cc in range(cb_n):
        rs = slice(cc * r, (cc + 1) * r)
        for j in range(hpc):
            pr, odd = j >> 1, j & 1
            w = wxr[pr][rs, odd * 2 * HD:odd * 2 * HD + HD]
            x = wxr[pr][rs, odd * 2 * HD + HD:(odd + 1) * 2 * HD]
            q = q_ref[rs, j * HD:(j + 1) * HD]
            kk = k_ref[rs, j * HD:(j + 1) * HD]
            n = n_ref[j]
            nb = n.astype(BF16)
            e = w.astype(F32) - jnp.dot(x, nb, preferred_element_type=F32)
            eb = e.astype(BF16)
            qk = lax.dot_general(q, kk, (((1,), (1,)), ((), ())),
                                 preferred_element_type=F32)
            aq = jnp.where(incl, beta * qk, 0.0).astype(BF16)
            o_ref[rs, j * HD:(j + 1) * HD] = (
                jnp.dot(q, nb, preferred_element_type=F32)
                + jnp.dot(aq, eb, preferred_element_type=F32)).astype(BF16)
            n_ref[j] = n + beta * lax.dot_general(
                kk, eb, (((0,), (0,)), ((), ())),
                preferred_element_type=F32)


def kernel(x, Wq, Wk, Wv, Wo):
    b, s, d = x.shape
    r = C * b
    nc = s // C
    beta = LR / b
    hpc = H // 2

    xt = x.transpose(1, 0, 2).reshape(s * b, d).astype(BF16)
    wqkv = jnp.concatenate([Wq.T, Wk.T, Wv.T], axis=1).astype(BF16)

    qkv = _matmul(xt, wqkv, BF16, bm=1024, bn=1024)

    cg_n = 8
    cb_n = 4
    solve = functools.partial(_solve_body, r=r, beta=beta, cg_n=cg_n)
    wx = pl.pallas_call(
        solve,
        grid=(H // 2, nc // cg_n),
        in_specs=[
            pl.BlockSpec((cg_n * r, 2 * HD),
                         lambda p, c: (c, H // 2 + p)),
            pl.BlockSpec((cg_n * r, 2 * HD),
                         lambda p, c: (c, H + p)),
        ],
        out_specs=pl.BlockSpec((cg_n * r, 4 * HD),
                               lambda p, c: (p * (nc // cg_n) + c, 0)),
        out_shape=jax.ShapeDtypeStruct((nc * (H // 2) * r, 4 * HD), BF16),
        compiler_params=pltpu.CompilerParams(
            dimension_semantics=("parallel", "parallel")),
        name="chunk_solve",
    )(qkv, qkv)

    sweep = functools.partial(_sweep_body, r=r, beta=beta, hpc=hpc, cb_n=cb_n)
    nb_c = nc // cb_n
    wx_spec = [
        pl.BlockSpec((cb_n * r, 4 * HD),
                     functools.partial(
                         lambda i, gg, c: ((gg * 4 + i) * nb_c + c, 0), i))
        for i in range(4)
    ]
    o = pl.pallas_call(
        sweep,
        grid=(2, nb_c),
        in_specs=wx_spec + [
            pl.BlockSpec((cb_n * r, hpc * HD), lambda gg, c: (c, gg)),
            pl.BlockSpec((cb_n * r, hpc * HD), lambda gg, c: (c, 2 + gg)),
        ],
        out_specs=pl.BlockSpec((cb_n * r, hpc * HD), lambda gg, c: (c, gg)),
        out_shape=jax.ShapeDtypeStruct((s * b, d), BF16),
        scratch_shapes=[pltpu.VMEM((hpc, HD, HD), F32)],
        compiler_params=pltpu.CompilerParams(
            dimension_semantics=("parallel", "arbitrary")),
        name="chunk_sweep",
    )(wx, wx, wx, wx, qkv, qkv)

    out = _matmul(o, Wo.T.astype(BF16), F32, bm=1024, bn=1024)
    return out.reshape(s, b, d).transpose(1, 0, 2)


# speedup vs baseline: 1.7549x; 1.6362x over previous
import functools

import jax
import jax.numpy as jnp
from jax import lax
from jax.experimental import pallas as pl
from jax.experimental.pallas import tpu as pltpu

H = 16
HD = 64
LR = 0.1
C = 32
F32 = jnp.float32


BF16 = jnp.bfloat16


def _mm_body(x_ref, w_ref, o_ref):
    o_ref[...] = jnp.dot(x_ref[...], w_ref[...],
                         preferred_element_type=F32).astype(o_ref.dtype)


def _matmul(x, w, out_dtype, bm=1024, bn=1024):
    m, k = x.shape
    _, n = w.shape
    return pl.pallas_call(
        _mm_body,
        grid=(m // bm, n // bn),
        in_specs=[
            pl.BlockSpec((bm, k), lambda i, j: (i, 0)),
            pl.BlockSpec((k, bn), lambda i, j: (0, j)),
        ],
        out_specs=pl.BlockSpec((bm, bn), lambda i, j: (i, j)),
        out_shape=jax.ShapeDtypeStruct((m, n), out_dtype),
        compiler_params=pltpu.CompilerParams(
            dimension_semantics=("parallel", "parallel")),
        name="proj_mm",
    )(x, w)


def _solve_body(k_ref, v_ref, wx_ref, *, r, beta, cg_n):
    r2 = 2 * r
    rsh = r.bit_length() - 1
    rows = lax.broadcasted_iota(jnp.int32, (r, r2), 0)
    cols = lax.broadcasted_iota(jnp.int32, (r, r2), 1)
    colm = cols & (r - 1)
    ident2 = jnp.where(colm == rows, 1.0, 0.0)
    lane_lo = cols < r
    rows2 = lax.broadcasted_iota(jnp.int32, (r2, r2), 0)
    cols2 = lax.broadcasted_iota(jnp.int32, (r2, r2), 1)
    same_head = (rows2 >> rsh) == (cols2 >> rsh)
    strict_d = same_head & (
        ((cols2 & (r - 1)) >> 2) < ((rows2 & (r - 1)) >> 2))
    zz = jnp.zeros((r, r), BF16)

    def bdiag(tp):
        top = jnp.where(lane_lo, tp, jnp.bfloat16(0))
        bot = jnp.where(lane_lo, jnp.bfloat16(0), tp)
        return jnp.concatenate([top, bot], axis=0)

    for ci in range(cg_n):
        v12 = v_ref[ci * r:(ci + 1) * r, :]
        k12 = k_ref[ci * r:(ci + 1) * r, :]
        ks = jnp.concatenate([k12[:, :HD], k12[:, HD:]], axis=0)
        gf = lax.dot_general(ks, ks, (((1,), (1,)), ((), ())),
                             preferred_element_type=F32)
        ad = jnp.where(strict_d, beta * gf, 0.0).astype(BF16)
        vkd = jnp.concatenate([
            jnp.concatenate([v12[:, :HD], k12[:, :HD], zz], axis=1),
            jnp.concatenate([zz, v12[:, HD:], k12[:, HD:]], axis=1),
        ], axis=0)
        t = ident2
        gsz, sh = 4, 2
        while gsz < r:
            tb = t.astype(BF16)
            td = bdiag(tb)
            u = jnp.dot(tb, ad, preferred_element_type=F32)
            u = jnp.dot(u.astype(BF16), td, preferred_element_type=F32)
            rg = rows >> sh
            cg = colm >> sh
            msub = ((rg & 1) == 1) & (cg == rg - 1)
            t = t - jnp.where(msub, u, 0.0)
            gsz, sh = gsz * 2, sh + 1
        wx_ref[ci * r:(ci + 1) * r, :] = jnp.dot(
            t.astype(BF16), vkd,
            preferred_element_type=F32).astype(BF16)


def _sweep_body(wx0, wx1, wx2, wx3, q_ref, k_ref, o_ref, n_ref,
                *, r, beta, hpc, cb_n):
    c = pl.program_id(1)

    @pl.when(c == 0)
    def _():
        n_ref[...] = jnp.zeros_like(n_ref)

    rows = lax.broadcasted_iota(jnp.int32, (r, r), 0)
    cols = lax.broadcasted_iota(jnp.int32, (r, r), 1)
    incl = (cols >> 2) <= (rows >> 2)
    wxr = (wx0, wx1, wx2, wx3)

    for cc in range(cb_n):
        rs = slice(cc * r, (cc + 1) * r)
        for j in range(hpc):
            pr, odd = j >> 1, j & 1
            w = wxr[pr][rs, odd * 2 * HD:odd * 2 * HD + HD]
            x = wxr[pr][rs, odd * 2 * HD + HD:(odd + 1) * 2 * HD]
            q = q_ref[rs, j * HD:(j + 1) * HD]
            kk = k_ref[rs, j * HD:(j + 1) * HD]
            n = n_ref[j]
            nb = n.astype(BF16)
            e = w.astype(F32) - jnp.dot(x, nb, preferred_element_type=F32)
            eb = e.astype(BF16)
            qk = lax.dot_general(q, kk, (((1,), (1,)), ((), ())),
                                 preferred_element_type=F32)
            aq = jnp.where(incl, beta * qk, 0.0).astype(BF16)
            o_ref[rs, j * HD:(j + 1) * HD] = (
                jnp.dot(q, nb, preferred_element_type=F32)
                + jnp.dot(aq, eb, preferred_element_type=F32)).astype(BF16)
            n_ref[j] = n + beta * lax.dot_general(
                kk, eb, (((0,), (0,)), ((), ())),
                preferred_element_type=F32)


def kernel(x, Wq, Wk, Wv, Wo):
    b, s, d = x.shape
    r = C * b
    nc = s // C
    beta = LR / b
    hpc = H // 2

    xt = x.transpose(1, 0, 2).reshape(s * b, d).astype(BF16)
    wqkv = jnp.concatenate([Wq.T, Wk.T, Wv.T], axis=1).astype(BF16)

    qkv = _matmul(xt, wqkv, BF16, bm=1024, bn=1024)

    cg_n = 8
    cb_n = 4
    solve = functools.partial(_solve_body, r=r, beta=beta, cg_n=cg_n)
    wx = pl.pallas_call(
        solve,
        grid=(H // 2, nc // cg_n),
        in_specs=[
            pl.BlockSpec((cg_n * r, 2 * HD),
                         lambda p, c: (c, H // 2 + p)),
            pl.BlockSpec((cg_n * r, 2 * HD),
                         lambda p, c: (c, H + p)),
        ],
        out_specs=pl.BlockSpec((cg_n * r, 4 * HD),
                               lambda p, c: (p * (nc // cg_n) + c, 0)),
        out_shape=jax.ShapeDtypeStruct((nc * (H // 2) * r, 4 * HD), BF16),
        compiler_params=pltpu.CompilerParams(
            dimension_semantics=("parallel", "parallel")),
        name="chunk_solve",
    )(qkv, qkv)

    sweep = functools.partial(_sweep_body, r=r, beta=beta, hpc=hpc, cb_n=cb_n)
    nb_c = nc // cb_n
    wx_spec = [
        pl.BlockSpec((cb_n * r, 4 * HD),
                     functools.partial(
                         lambda i, gg, c: ((gg * 4 + i) * nb_c + c, 0), i))
        for i in range(4)
    ]
    o = pl.pallas_call(
        sweep,
        grid=(2, nb_c),
        in_specs=wx_spec + [
            pl.BlockSpec((cb_n * r, hpc * HD), lambda gg, c: (c, gg)),
            pl.BlockSpec((cb_n * r, hpc * HD), lambda gg, c: (c, 2 + gg)),
        ],
        out_specs=pl.BlockSpec((cb_n * r, hpc * HD), lambda gg, c: (c, gg)),
        out_shape=jax.ShapeDtypeStruct((s * b, d), BF16),
        scratch_shapes=[pltpu.VMEM((hpc, HD, HD), F32)],
        compiler_params=pltpu.CompilerParams(
            dimension_semantics=("parallel", "arbitrary")),
        name="chunk_sweep",
    )(wx, wx, wx, wx, qkv, qkv)

    out = _matmul(o, Wo.T.astype(BF16), F32, bm=1024, bn=1024)
    return out.reshape(s, b, d).transpose(1, 0, 2)
